# baseline (speedup 1.0000x reference)
import functools

import jax
import jax.numpy as jnp
import numpy as np
from jax import lax
from jax.experimental import pallas as pl
from jax.experimental.pallas import tpu as pltpu
from jax.experimental.pallas import tpu_sc as plsc

B = 16384
N_OTHER = 30
N_TAGS = 20
N_PUBS = 5
N_COLS = N_OTHER + N_TAGS + N_PUBS
TAG_VOCAB = 100000
PUB_VOCAB = 100000
TAG_DIM = 128
PUB_DIM = 32
FEAT = 192

NC = 2
NS = 16
NW = NC * NS
SPW = B // NW
CT = 16
NCHUNK = SPW // CT
NPAIR = NCHUNK // 2
TROWS = CT * N_TAGS
PROWS = CT * N_PUBS

W_DELTA = np.float32((0.1 - 1.0) / (TAG_VOCAB - 1))


def _pool_body(x3_hbm, tag_table, pub_table, feat_out,
               xi_b, tidx_b, pidx_b, trows_b, prows_b, fout_b,
               gsem_b, osem_b):
  i32 = jnp.int32
  cid = lax.axis_index("c")
  sid = lax.axis_index("s")
  wid = sid * i32(NC) + cid
  base = wid * i32(SPW)
  lanes = lax.iota(jnp.int32, 16)
  col_t0 = 30 + lanes
  col_t1 = jnp.where(lanes < 4, 46 + lanes, 46)
  col_p = jnp.where(lanes < 5, 50 + lanes, 54)
  col_o0 = lanes
  col_o1 = jnp.where(lanes < 14, 16 + lanes, 29)

  for fv in fout_b:
    def pad_body(si, _, fv=fv):
      fv[si, pl.ds(FEAT - 16, 16)] = jnp.zeros((16,), jnp.float32)
      return i32(0)
    lax.fori_loop(i32(0), i32(CT), pad_body, i32(0))

  def stage1(ci, owait, b):
    xi_v, tidx_v, pidx_v, trows_v = xi_b[b], tidx_b[b], pidx_b[b], trows_b[b]
    fout_v = fout_b[b]
    s0 = base + ci * i32(CT)
    pltpu.sync_copy(x3_hbm.at[pl.ds(s0, CT)], xi_v)

    @pl.when(owait)
    def _():
      pltpu.make_async_copy(feat_out.at[pl.ds(0, CT)], fout_v, osem_b[b]).wait()

    def extract_body(si, _):
      tb = si * i32(N_TAGS)
      pb = si * i32(N_PUBS)
      row = jnp.full((16,), si, jnp.int32)
      tg0 = plsc.load_gather(xi_v, [row, col_t0])
      tg1 = plsc.load_gather(xi_v, [row, col_t1])
      pg = plsc.load_gather(xi_v, [row, col_p])
      og0 = plsc.load_gather(xi_v, [row, col_o0]).astype(jnp.float32)
      og1 = plsc.load_gather(xi_v, [row, col_o1]).astype(jnp.float32)
      tidx_v[pl.ds(tb, 16)] = tg0
      tidx_v[pl.ds(tb + i32(16), 16)] = tg1
      pidx_b[b][pl.ds(pb, 16)] = pg
      fout_v[si, pl.ds(0, 16)] = og0
      fout_v[si, pl.ds(16, 16)] = og1
      return i32(0)

    lax.fori_loop(i32(0), i32(CT), extract_body, i32(0))
    for j in range(TROWS // 128):
      pltpu.async_copy(
          tag_table.at[tidx_v.at[pl.ds(j * 128, 128)]],
          trows_v.at[pl.ds(j * 128, 128)], gsem_b[b])
    rem = TROWS % 128
    if rem:
      pltpu.async_copy(
          tag_table.at[tidx_v.at[pl.ds(TROWS - rem, rem)]],
          trows_v.at[pl.ds(TROWS - rem, rem)], gsem_b[b])
    pltpu.async_copy(
        pub_table.at[pidx_v.at[pl.ds(0, PROWS)]],
        prows_b[b].at[pl.ds(0, PROWS)], gsem_b[b])

  def stage2(ci, b):
    tidx_v, pidx_v, trows_v = tidx_b[b], pidx_b[b], trows_b[b]
    prows_v, fout_v = prows_b[b], fout_b[b]
    s0 = base + ci * i32(CT)
    pltpu.make_async_copy(tag_table.at[pl.ds(0, TROWS)], trows_v,
                          gsem_b[b]).wait()
    pltpu.make_async_copy(pub_table.at[pl.ds(0, PROWS)],
                          prows_v.at[pl.ds(0, PROWS)], gsem_b[b]).wait()

    def sample_body(si, _):
      tb = si * jnp.int32(N_TAGS)
      va = tidx_v[pl.ds(tb, 16)]
      vb = tidx_v[pl.ds(tb + jnp.int32(4), 16)]
      wa = jnp.where(va != 0, 1.0 + va.astype(jnp.float32) * W_DELTA, 0.0)
      wb = jnp.where(vb != 0, 1.0 + vb.astype(jnp.float32) * W_DELTA, 0.0)
      tacc = [jnp.zeros((16,), jnp.float32) for _ in range(TAG_DIM // 16)]
      tn = jnp.float32(0)
      for t in range(N_TAGS):
        w = wa[t] if t < 16 else wb[t - 4]
        tn = tn + jnp.where(w != 0, jnp.float32(1.0), jnp.float32(0.0))
        row = tb + jnp.int32(t)
        for k in range(TAG_DIM // 16):
          tacc[k] = tacc[k] + trows_v[row, pl.ds(k * 16, 16)] * w
      tden = jnp.maximum(tn, 1.0)
      for k in range(TAG_DIM // 16):
        fout_v[si, pl.ds(N_OTHER + k * 16, 16)] = tacc[k] / tden
      pb = si * jnp.int32(N_PUBS)
      pv = pidx_v[pl.ds(pb, 16)]
      pw = jnp.where(pv != 0, jnp.float32(1.0), jnp.float32(0.0))
      pacc = [jnp.zeros((16,), jnp.float32) for _ in range(PUB_DIM // 16)]
      pn = jnp.float32(0)
      for t in range(N_PUBS):
        w = pw[t]
        pn = pn + w
        row = pb + jnp.int32(t)
        for k in range(PUB_DIM // 16):
          pacc[k] = pacc[k] + prows_v[row, pl.ds(k * 16, 16)] * w
      pden = jnp.maximum(pn, 1.0)
      for k in range(PUB_DIM // 16):
        fout_v[si, pl.ds(N_OTHER + TAG_DIM + k * 16, 16)] = pacc[k] / pden
      return jnp.int32(0)

    lax.fori_loop(i32(0), i32(CT), sample_body, i32(0))
    pltpu.async_copy(fout_v, feat_out.at[pl.ds(s0, CT)], osem_b[b])

  stage1(i32(0), jnp.bool_(False), 0)

  def pair_body(k, _):
    c0 = i32(2) * k
    stage1(c0 + i32(1), k > i32(0), 1)
    stage2(c0, 0)

    @pl.when(k < i32(NPAIR - 1))
    def _():
      stage1(c0 + i32(2), jnp.bool_(True), 0)

    stage2(c0 + i32(1), 1)
    return i32(0)

  lax.fori_loop(i32(0), i32(NPAIR), pair_body, i32(0))
  pltpu.make_async_copy(feat_out.at[pl.ds(0, CT)], fout_b[0], osem_b[0]).wait()
  pltpu.make_async_copy(feat_out.at[pl.ds(0, CT)], fout_b[1], osem_b[1]).wait()


@jax.jit
def _pool(x3, tag_table, pub_table):
  mesh = plsc.VectorSubcoreMesh(core_axis_name="c", subcore_axis_name="s")
  return pl.kernel(
      _pool_body,
      out_type=jax.ShapeDtypeStruct((B, FEAT), jnp.float32),
      mesh=mesh,
      compiler_params=pltpu.CompilerParams(use_tc_tiling_on_sc=False,
                                           needs_layout_passes=False),
      scratch_types=[
          [pltpu.VMEM((CT, N_COLS), jnp.int32)] * 2,
          [pltpu.VMEM((TROWS + 16,), jnp.int32)] * 2,
          [pltpu.VMEM((PROWS + 16,), jnp.int32)] * 2,
          [pltpu.VMEM((TROWS, TAG_DIM), jnp.float32)] * 2,
          [pltpu.VMEM((PROWS, PUB_DIM), jnp.float32)] * 2,
          [pltpu.VMEM((CT, FEAT), jnp.float32)] * 2,
          [pltpu.SemaphoreType.DMA] * 2,
          [pltpu.SemaphoreType.DMA] * 2,
      ],
  )(x3, tag_table, pub_table)


MLP_BLK = 2048


def _dot(a, b):
  return jax.lax.dot(a, b, precision=jax.lax.Precision.DEFAULT,
                     preferred_element_type=jnp.float32)


def _mlp_body(feat_ref, W1p, b1, W2, b2, W3, b3, W4, b4, W5, b5, out_ref):
  h = _dot(feat_ref[...], W1p[...]) + b1[...]
  h = jnp.maximum(h, 0.0)
  h = jnp.maximum(_dot(h, W2[...]) + b2[...], 0.0)
  h = jnp.maximum(_dot(h, W3[...]) + b3[...], 0.0)
  h = jnp.maximum(_dot(h, W4[...]) + b4[...], 0.0)
  z = _dot(h, W5[...]) + b5[...]
  out_ref[...] = jax.nn.sigmoid(z)


@jax.jit
def _mlp(feat, W1, b1, W2, b2, W3, b3, W4, b4, W5, b5):
  nblk = B // MLP_BLK
  z = np.int32(0)
  bspec = lambda d: pl.BlockSpec((MLP_BLK, d), lambda i: (i, z))
  wspec = lambda r, c: pl.BlockSpec((r, c), lambda i: (z, z))
  vspec = lambda d: pl.BlockSpec((d,), lambda i: (z,))
  W1p = jnp.concatenate([W1, jnp.zeros((2, 256), jnp.float32)], axis=0)
  return pl.pallas_call(
      _mlp_body,
      grid=(nblk,),
      in_specs=[
          bspec(FEAT),
          wspec(FEAT, 256), vspec(256),
          wspec(256, 128), vspec(128),
          wspec(128, 64), vspec(64),
          wspec(64, 32), vspec(32),
          wspec(32, 1), vspec(1),
      ],
      out_specs=pl.BlockSpec((MLP_BLK, 1), lambda i: (i, z)),
      out_shape=jax.ShapeDtypeStruct((B, 1), jnp.float32),
  )(feat, W1p, b1, W2, b2, W3, b3, W4, b4, W5, b5)


def kernel(x, tag_table, pub_table, W1, b1, W2, b2, W3, b3, W4, b4, W5, b5):
  x32 = x.astype(jnp.int32)
  feat = _pool(x32, tag_table, pub_table)
  return _mlp(feat, W1, b1, W2, b2, W3, b3, W4, b4, W5, b5)


# speedup vs baseline: 1.1335x; 1.1335x over previous
import functools

import jax
import jax.numpy as jnp
import numpy as np
from jax import lax
from jax.experimental import pallas as pl
from jax.experimental.pallas import tpu as pltpu
from jax.experimental.pallas import tpu_sc as plsc

B = 16384
N_OTHER = 30
N_TAGS = 20
N_PUBS = 5
N_COLS = N_OTHER + N_TAGS + N_PUBS
TAG_VOCAB = 100000
PUB_VOCAB = 100000
TAG_DIM = 128
PUB_DIM = 32
FEAT = 192

NC = 2
NS = 16
NW = NC * NS
SPW = B // NW
CT = 16
NCHUNK = SPW // CT
NPAIR = NCHUNK // 2
TROWS = CT * N_TAGS
PROWS = CT * N_PUBS

W_DELTA = np.float32((0.1 - 1.0) / (TAG_VOCAB - 1))


def _pool_body(x3_hbm, tag_table, pub_table, feat_out,
               xi_b, tidx_b, pidx_b, trows_b, prows_b, fout_b,
               gsem_b, osem_b):
  i32 = jnp.int32
  cid = lax.axis_index("c")
  sid = lax.axis_index("s")
  wid = sid * i32(NC) + cid
  base = wid * i32(SPW)
  lanes = lax.iota(jnp.int32, 16)
  col_t0 = 60 + 2 * lanes
  col_t1 = jnp.where(lanes < 4, 92 + 2 * lanes, 92)
  col_p = jnp.where(lanes < 5, 100 + 2 * lanes, 108)
  col_o0 = 2 * lanes
  col_o1 = jnp.where(lanes < 14, 32 + 2 * lanes, 58)

  for fv in fout_b:
    def pad_body(si, _, fv=fv):
      fv[si, pl.ds(FEAT - 16, 16)] = jnp.zeros((16,), jnp.float32)
      return i32(0)
    lax.fori_loop(i32(0), i32(CT), pad_body, i32(0))

  def stage1(ci, owait, b):
    xi_v, tidx_v, pidx_v, trows_v = xi_b[b], tidx_b[b], pidx_b[b], trows_b[b]
    fout_v = fout_b[b]
    s0 = base + ci * i32(CT)
    pltpu.sync_copy(x3_hbm.at[pl.ds(s0, CT)], xi_v)

    @pl.when(owait)
    def _():
      pltpu.make_async_copy(feat_out.at[pl.ds(0, CT)], fout_v, osem_b[b]).wait()

    def extract_body(si, _):
      tb = si * i32(N_TAGS)
      pb = si * i32(N_PUBS)
      row = jnp.full((16,), si, jnp.int32)
      tg0 = plsc.load_gather(xi_v, [row, col_t0])
      tg1 = plsc.load_gather(xi_v, [row, col_t1])
      pg = plsc.load_gather(xi_v, [row, col_p])
      og0 = plsc.load_gather(xi_v, [row, col_o0]).astype(jnp.float32)
      og1 = plsc.load_gather(xi_v, [row, col_o1]).astype(jnp.float32)
      tidx_v[pl.ds(tb, 16)] = tg0
      tidx_v[pl.ds(tb + i32(16), 16)] = tg1
      pidx_b[b][pl.ds(pb, 16)] = pg
      fout_v[si, pl.ds(0, 16)] = og0
      fout_v[si, pl.ds(16, 16)] = og1
      return i32(0)

    lax.fori_loop(i32(0), i32(CT), extract_body, i32(0))
    for j in range(TROWS // 128):
      pltpu.async_copy(
          tag_table.at[tidx_v.at[pl.ds(j * 128, 128)]],
          trows_v.at[pl.ds(j * 128, 128)], gsem_b[b])
    rem = TROWS % 128
    if rem:
      pltpu.async_copy(
          tag_table.at[tidx_v.at[pl.ds(TROWS - rem, rem)]],
          trows_v.at[pl.ds(TROWS - rem, rem)], gsem_b[b])
    pltpu.async_copy(
        pub_table.at[pidx_v.at[pl.ds(0, PROWS)]],
        prows_b[b].at[pl.ds(0, PROWS)], gsem_b[b])

  def stage2(ci, b):
    tidx_v, pidx_v, trows_v = tidx_b[b], pidx_b[b], trows_b[b]
    prows_v, fout_v = prows_b[b], fout_b[b]
    s0 = base + ci * i32(CT)
    pltpu.make_async_copy(tag_table.at[pl.ds(0, TROWS)], trows_v,
                          gsem_b[b]).wait()
    pltpu.make_async_copy(pub_table.at[pl.ds(0, PROWS)],
                          prows_v.at[pl.ds(0, PROWS)], gsem_b[b]).wait()

    def sample_body(si, _):
      tb = si * jnp.int32(N_TAGS)
      va = tidx_v[pl.ds(tb, 16)]
      vb = tidx_v[pl.ds(tb + jnp.int32(4), 16)]
      wa = jnp.where(va != 0, 1.0 + va.astype(jnp.float32) * W_DELTA, 0.0)
      wb = jnp.where(vb != 0, 1.0 + vb.astype(jnp.float32) * W_DELTA, 0.0)
      tacc = [jnp.zeros((16,), jnp.float32) for _ in range(TAG_DIM // 16)]
      tn = jnp.float32(0)
      for t in range(N_TAGS):
        w = wa[t] if t < 16 else wb[t - 4]
        tn = tn + jnp.where(w != 0, jnp.float32(1.0), jnp.float32(0.0))
        row = tb + jnp.int32(t)
        for k in range(TAG_DIM // 16):
          tacc[k] = tacc[k] + trows_v[row, pl.ds(k * 16, 16)] * w
      tden = jnp.maximum(tn, 1.0)
      for k in range(TAG_DIM // 16):
        fout_v[si, pl.ds(N_OTHER + k * 16, 16)] = tacc[k] / tden
      pb = si * jnp.int32(N_PUBS)
      pv = pidx_v[pl.ds(pb, 16)]
      pw = jnp.where(pv != 0, jnp.float32(1.0), jnp.float32(0.0))
      pacc = [jnp.zeros((16,), jnp.float32) for _ in range(PUB_DIM // 16)]
      pn = jnp.float32(0)
      for t in range(N_PUBS):
        w = pw[t]
        pn = pn + w
        row = pb + jnp.int32(t)
        for k in range(PUB_DIM // 16):
          pacc[k] = pacc[k] + prows_v[row, pl.ds(k * 16, 16)] * w
      pden = jnp.maximum(pn, 1.0)
      for k in range(PUB_DIM // 16):
        fout_v[si, pl.ds(N_OTHER + TAG_DIM + k * 16, 16)] = pacc[k] / pden
      return jnp.int32(0)

    lax.fori_loop(i32(0), i32(CT), sample_body, i32(0))
    pltpu.async_copy(fout_v, feat_out.at[pl.ds(s0, CT)], osem_b[b])

  stage1(i32(0), jnp.bool_(False), 0)

  def pair_body(k, _):
    c0 = i32(2) * k
    stage1(c0 + i32(1), k > i32(0), 1)
    stage2(c0, 0)

    @pl.when(k < i32(NPAIR - 1))
    def _():
      stage1(c0 + i32(2), jnp.bool_(True), 0)

    stage2(c0 + i32(1), 1)
    return i32(0)

  lax.fori_loop(i32(0), i32(NPAIR), pair_body, i32(0))
  pltpu.make_async_copy(feat_out.at[pl.ds(0, CT)], fout_b[0], osem_b[0]).wait()
  pltpu.make_async_copy(feat_out.at[pl.ds(0, CT)], fout_b[1], osem_b[1]).wait()


@jax.jit
def _pool(x3, tag_table, pub_table):
  mesh = plsc.VectorSubcoreMesh(core_axis_name="c", subcore_axis_name="s")
  return pl.kernel(
      _pool_body,
      out_type=jax.ShapeDtypeStruct((B, FEAT), jnp.float32),
      mesh=mesh,
      compiler_params=pltpu.CompilerParams(use_tc_tiling_on_sc=False,
                                           needs_layout_passes=False),
      scratch_types=[
          [pltpu.VMEM((CT, 2 * N_COLS), jnp.int32)] * 2,
          [pltpu.VMEM((TROWS + 16,), jnp.int32)] * 2,
          [pltpu.VMEM((PROWS + 16,), jnp.int32)] * 2,
          [pltpu.VMEM((TROWS, TAG_DIM), jnp.float32)] * 2,
          [pltpu.VMEM((PROWS, PUB_DIM), jnp.float32)] * 2,
          [pltpu.VMEM((CT, FEAT), jnp.float32)] * 2,
          [pltpu.SemaphoreType.DMA] * 2,
          [pltpu.SemaphoreType.DMA] * 2,
      ],
  )(x3, tag_table, pub_table)


MLP_BLK = 2048


def _dot(a, b):
  return jax.lax.dot(a, b, precision=jax.lax.Precision.DEFAULT,
                     preferred_element_type=jnp.float32)


def _mlp_body(feat_ref, W1p, b1, W2, b2, W3, b3, W4, b4, W5, b5, out_ref):
  h = _dot(feat_ref[...], W1p[...]) + b1[...]
  h = jnp.maximum(h, 0.0)
  h = jnp.maximum(_dot(h, W2[...]) + b2[...], 0.0)
  h = jnp.maximum(_dot(h, W3[...]) + b3[...], 0.0)
  h = jnp.maximum(_dot(h, W4[...]) + b4[...], 0.0)
  z = _dot(h, W5[...]) + b5[...]
  out_ref[...] = jax.nn.sigmoid(z)


@jax.jit
def _mlp(feat, W1, b1, W2, b2, W3, b3, W4, b4, W5, b5):
  nblk = B // MLP_BLK
  z = np.int32(0)
  bspec = lambda d: pl.BlockSpec((MLP_BLK, d), lambda i: (i, z))
  wspec = lambda r, c: pl.BlockSpec((r, c), lambda i: (z, z))
  vspec = lambda d: pl.BlockSpec((d,), lambda i: (z,))
  W1p = jnp.concatenate([W1, jnp.zeros((2, 256), jnp.float32)], axis=0)
  return pl.pallas_call(
      _mlp_body,
      grid=(nblk,),
      in_specs=[
          bspec(FEAT),
          wspec(FEAT, 256), vspec(256),
          wspec(256, 128), vspec(128),
          wspec(128, 64), vspec(64),
          wspec(64, 32), vspec(32),
          wspec(32, 1), vspec(1),
      ],
      out_specs=pl.BlockSpec((MLP_BLK, 1), lambda i: (i, z)),
      out_shape=jax.ShapeDtypeStruct((B, 1), jnp.float32),
  )(feat, W1p, b1, W2, b2, W3, b3, W4, b4, W5, b5)


def kernel(x, tag_table, pub_table, W1, b1, W2, b2, W3, b3, W4, b4, W5, b5):
  x32 = jax.lax.bitcast_convert_type(x, jnp.int32).reshape(B, 2 * N_COLS)
  feat = _pool(x32, tag_table, pub_table)
  return _mlp(feat, W1, b1, W2, b2, W3, b3, W4, b4, W5, b5)
